# Initial kernel scaffold; baseline (speedup 1.0000x reference)
import jax
import jax.numpy as jnp
from jax.experimental import pallas as pl


def kernel(x, edge_index, edge_weight, W0, b0, W1, b1, s0, sb0, dk0, dkb0, s1, sb1, dk1, dkb1):
    raise NotImplementedError("write your pallas kernel here")


# speedup vs baseline: 1.8023x; 1.8023x over previous
import functools

import jax
import jax.numpy as jnp
from jax import lax
from jax.experimental import pallas as pl

N = 10000
F_IN = 256
K = 20
GAMMA = 0.01

ROW_BLK = 200


def _knn_body(a_r_ref, a_all_ref, vals_ref, idx_ref):
    i = pl.program_id(0)
    sims = lax.dot_general(a_r_ref[...], a_all_ref[...], (((1,), (1,)), ((), ())),
                           preferred_element_type=jnp.float32)
    col = lax.broadcasted_iota(jnp.int32, sims.shape, 1)
    rowid = lax.broadcasted_iota(jnp.int32, sims.shape, 0) + i * ROW_BLK
    sims = jnp.where(col == rowid, 0.0, sims)

    vlist = []
    ilist = []
    for _ in range(K):
        mx = jnp.max(sims, axis=1, keepdims=True)
        j = jnp.min(jnp.where(sims == mx, col, N), axis=1, keepdims=True)
        vlist.append(mx)
        ilist.append(j)
        sims = jnp.where(col == j, -1.0, sims)
    vals_ref[...] = jnp.concatenate(vlist, axis=1)
    idx_ref[...] = jnp.concatenate(ilist, axis=1)


@jax.jit
def _knn_topk(a):
    grid = N // ROW_BLK
    vals, idx = pl.pallas_call(
        _knn_body,
        grid=(grid,),
        in_specs=[
            pl.BlockSpec((ROW_BLK, F_IN), lambda i: (i, 0)),
            pl.BlockSpec((N, F_IN), lambda i: (0, 0)),
        ],
        out_specs=[
            pl.BlockSpec((ROW_BLK, K), lambda i: (i, 0)),
            pl.BlockSpec((ROW_BLK, K), lambda i: (i, 0)),
        ],
        out_shape=[
            jax.ShapeDtypeStruct((N, K), jnp.float32),
            jax.ShapeDtypeStruct((N, K), jnp.int32),
        ],
    )(a, a)
    return vals, idx


def _gcn_conv(h, row, col, w, W, b, n):
    m = h @ W
    loop = jnp.arange(n)
    row2 = jnp.concatenate([row, loop])
    col2 = jnp.concatenate([col, loop])
    w2 = jnp.concatenate([w.astype(m.dtype), jnp.ones((n,), m.dtype)])
    deg = jax.ops.segment_sum(w2, col2, num_segments=n)
    dinv = jnp.where(deg > 0, deg ** -0.5, 0.0)
    norm = dinv[row2] * w2 * dinv[col2]
    out = jax.ops.segment_sum(norm[:, None] * m[row2], col2, num_segments=n)
    return out + b


def kernel(x, edge_index, edge_weight, W0, b0, W1, b1, s0, sb0, dk0, dkb0, s1, sb1, dk1, dkb1):
    xb = (x != 0).astype(jnp.float32)
    a = xb / jnp.linalg.norm(xb, axis=1, keepdims=True)
    vals, idx = _knn_topk(a)
    krow = jnp.repeat(jnp.arange(N), K)
    kcol = idx.reshape(-1)
    kw = vals.reshape(-1)

    n = N
    h = x
    for (W, b, sc, scb, dk, dkb) in ((W0, b0, s0, sb0, dk0, dkb0), (W1, b1, s1, sb1, dk1, dkb1)):
        s = jax.nn.sigmoid(h @ sc + scb)
        Dk = h @ dk + dkb
        tmp = h @ W + b
        tmp_knn = _gcn_conv(h, krow, kcol, kw, W, b, n)
        g = _gcn_conv(h, edge_index[0], edge_index[1], edge_weight, W, b, n)
        h = s * g + (1.0 - s) * tmp_knn + GAMMA * Dk * tmp
    return h


# speedup vs baseline: 4.6758x; 2.5943x over previous
import functools

import jax
import jax.numpy as jnp
from jax import lax
from jax.experimental import pallas as pl
from jax.experimental.pallas import tpu as pltpu
from jax.experimental.pallas import tpu_sc as plsc

N = 10000
NPAD = 10008
F_IN = 256
HID = 128
OUT = 64
K = 20
GAMMA = 0.01

ROW_BLK = 200
RB = 200
NC, NS = 2, 16
NW = NC * NS
CH = 128

E_G = 160000
E_K = N * K
EPAD_G = ((E_G + NW * CH - 1) // (NW * CH)) * NW * CH
EPAD_K = ((E_K + NW * CH - 1) // (NW * CH)) * NW * CH
PT_G = EPAD_G // NW
PT_K = EPAD_K // NW


def _knn_body(a_r_ref, a_all_ref, vals_ref, idx_ref):
    i = pl.program_id(0)
    sims = lax.dot_general(a_r_ref[...], a_all_ref[...], (((1,), (1,)), ((), ())),
                           preferred_element_type=jnp.float32)
    col = lax.broadcasted_iota(jnp.int32, sims.shape, 1)
    rowid = lax.broadcasted_iota(jnp.int32, sims.shape, 0) + i * ROW_BLK
    sims = jnp.where(col == rowid, 0.0, sims)
    vlist, ilist = [], []
    for _ in range(K):
        mx = jnp.max(sims, axis=1, keepdims=True)
        j = jnp.min(jnp.where(sims == mx, col, N), axis=1, keepdims=True)
        vlist.append(mx)
        ilist.append(j)
        sims = jnp.where(col == j, -1.0, sims)
    vals_ref[...] = jnp.concatenate(vlist, axis=1)
    idx_ref[...] = jnp.concatenate(ilist, axis=1)


def _knn_topk(a):
    return pl.pallas_call(
        _knn_body,
        grid=(N // ROW_BLK,),
        in_specs=[
            pl.BlockSpec((ROW_BLK, F_IN), lambda i: (i, 0)),
            pl.BlockSpec((N, F_IN), lambda i: (0, 0)),
        ],
        out_specs=[
            pl.BlockSpec((ROW_BLK, K), lambda i: (i, 0)),
            pl.BlockSpec((ROW_BLK, K), lambda i: (i, 0)),
        ],
        out_shape=[
            jax.ShapeDtypeStruct((N, K), jnp.float32),
            jax.ShapeDtypeStruct((N, K), jnp.int32),
        ],
    )(a, a)


def _make_agg(F, per_tile):
    nchunks = per_tile // CH

    def body(msc, src, dst, w, zerosF, out, sbuf, dbuf, wbuf, rows, acc, sem):
        c = lax.axis_index("c")
        s = lax.axis_index("s")
        wid = s * NC + c
        base = wid * per_tile

        @pl.when(s == 0)
        def _():
            pltpu.sync_copy(zerosF, acc)
        plsc.subcore_barrier()

        def step(i, carry):
            off = base + i * CH
            pltpu.sync_copy(src.at[pl.ds(off, CH)], sbuf)
            pltpu.sync_copy(dst.at[pl.ds(off, CH)], dbuf)
            pltpu.sync_copy(w.at[pl.ds(off, CH)], wbuf)
            pltpu.async_copy(msc.at[sbuf], rows, sem).wait()

            def scale(e, cc):
                wv = plsc.load_gather(wbuf, [jnp.full((16,), e, jnp.int32)])
                for f in range(F // 16):
                    sl = pl.ds(f * 16, 16)
                    rows[e, sl] = rows[e, sl] * wv
                return cc
            lax.fori_loop(0, CH, scale, 0)
            pltpu.sync_copy(rows, acc.at[dbuf], add=True)
            return carry
        lax.fori_loop(0, nchunks, step, 0)
        plsc.subcore_barrier()

        @pl.when(s == 0)
        def _():
            pltpu.sync_copy(acc, out.at[c])

    mesh = plsc.VectorSubcoreMesh(core_axis_name="c", subcore_axis_name="s")

    def call(msc, src, dst, w, zerosF):
        f = pl.kernel(
            body,
            out_type=jax.ShapeDtypeStruct((NC, NPAD, F), jnp.float32),
            mesh=mesh,
            scratch_types=[
                pltpu.VMEM((CH,), jnp.int32),
                pltpu.VMEM((CH,), jnp.int32),
                pltpu.VMEM((CH,), jnp.float32),
                pltpu.VMEM((CH, F), jnp.float32),
                pltpu.VMEM_SHARED((NPAD, F), jnp.float32),
                pltpu.SemaphoreType.DMA,
            ],
            compiler_params=pltpu.CompilerParams(
                needs_layout_passes=False, use_tc_tiling_on_sc=False),
        )
        return f(msc, src, dst, w, zerosF)

    return call


_agg_hid = _make_agg(HID, PT_G)
_agg_hid_k = _make_agg(HID, PT_K)
_agg_out = _make_agg(OUT, PT_G)
_agg_out_k = _make_agg(OUT, PT_K)


def _dinv_body(dg_ref, dk_ref, og_ref, ok_ref):
    dg = dg_ref[0, :, 0:1] + dg_ref[1, :, 0:1] + 1.0
    dk = dk_ref[0, :, 0:1] + dk_ref[1, :, 0:1] + 1.0
    og_ref[...] = jnp.where(dg > 0, lax.rsqrt(dg), 0.0)
    ok_ref[...] = jnp.where(dk > 0, lax.rsqrt(dk), 0.0)


def _dinv_call(degg, degk):
    return pl.pallas_call(
        _dinv_body,
        grid=(N // RB,),
        in_specs=[
            pl.BlockSpec((NC, RB, HID), lambda i: (0, i, 0)),
            pl.BlockSpec((NC, RB, HID), lambda i: (0, i, 0)),
        ],
        out_specs=[
            pl.BlockSpec((RB, 1), lambda i: (i, 0)),
            pl.BlockSpec((RB, 1), lambda i: (i, 0)),
        ],
        out_shape=[
            jax.ShapeDtypeStruct((N, 1), jnp.float32),
            jax.ShapeDtypeStruct((N, 1), jnp.float32),
        ],
    )(degg, degk)


def _make_layer_a(Fin1, F):
    def body(h_ref, w_ref, dg_ref, dk_ref, ma_ref, mg_ref, mk_ref):
        ma = jnp.dot(h_ref[...], w_ref[...], preferred_element_type=jnp.float32)
        ma_ref[...] = ma
        m = ma[:, :F]
        mg_ref[...] = m * dg_ref[...]
        mk_ref[...] = m * dk_ref[...]

    def call(h_aug, w_aug, dinvg, dinvk):
        return pl.pallas_call(
            body,
            grid=(N // RB,),
            in_specs=[
                pl.BlockSpec((RB, Fin1), lambda i: (i, 0)),
                pl.BlockSpec((Fin1, F + 2), lambda i: (0, 0)),
                pl.BlockSpec((RB, 1), lambda i: (i, 0)),
                pl.BlockSpec((RB, 1), lambda i: (i, 0)),
            ],
            out_specs=[
                pl.BlockSpec((RB, F + 2), lambda i: (i, 0)),
                pl.BlockSpec((RB, F), lambda i: (i, 0)),
                pl.BlockSpec((RB, F), lambda i: (i, 0)),
            ],
            out_shape=[
                jax.ShapeDtypeStruct((N, F + 2), jnp.float32),
                jax.ShapeDtypeStruct((N, F), jnp.float32),
                jax.ShapeDtypeStruct((N, F), jnp.float32),
            ],
        )(h_aug, w_aug, dinvg, dinvk)

    return call


_layer_a_1 = _make_layer_a(F_IN + 1, HID)
_layer_a_2 = _make_layer_a(HID + 1, OUT)


def _make_combine(F):
    def body(ma_ref, ag_ref, ak_ref, dg_ref, dk_ref, b_ref, out_ref):
        ma = ma_ref[...]
        m = ma[:, :F]
        s = jax.nn.sigmoid(ma[:, F:F + 1])
        hdk = ma[:, F + 1:F + 2]
        dg = dg_ref[...]
        dk = dk_ref[...]
        b = b_ref[...]
        g = dg * (ag_ref[0] + ag_ref[1]) + dg * dg * m + b
        kn = dk * (ak_ref[0] + ak_ref[1]) + dk * dk * m + b
        out_ref[...] = s * g + (1.0 - s) * kn + GAMMA * hdk * (m + b)

    def call(maug, aggg, aggk, dinvg, dinvk, b):
        return pl.pallas_call(
            body,
            grid=(N // RB,),
            in_specs=[
                pl.BlockSpec((RB, F + 2), lambda i: (i, 0)),
                pl.BlockSpec((NC, RB, F), lambda i: (0, i, 0)),
                pl.BlockSpec((NC, RB, F), lambda i: (0, i, 0)),
                pl.BlockSpec((RB, 1), lambda i: (i, 0)),
                pl.BlockSpec((RB, 1), lambda i: (i, 0)),
                pl.BlockSpec((1, F), lambda i: (0, 0)),
            ],
            out_specs=pl.BlockSpec((RB, F), lambda i: (i, 0)),
            out_shape=jax.ShapeDtypeStruct((N, F), jnp.float32),
        )(maug, aggg, aggk, dinvg, dinvk, b)

    return call


_combine_1 = _make_combine(HID)
_combine_2 = _make_combine(OUT)


def _pad1(a, n, val):
    return jnp.concatenate([a, jnp.full((n - a.shape[0],), val, a.dtype)])


def kernel(x, edge_index, edge_weight, W0, b0, W1, b1, s0, sb0, dk0, dkb0, s1, sb1, dk1, dkb1):
    xb = (x != 0).astype(jnp.float32)
    a = xb / jnp.linalg.norm(xb, axis=1, keepdims=True)
    vals, idx = _knn_topk(a)

    ei = edge_index.astype(jnp.int32)
    src_g = _pad1(ei[0], EPAD_G, 0)
    dst_g = _pad1(ei[1], EPAD_G, N)
    w_g = _pad1(edge_weight.astype(jnp.float32), EPAD_G, 0.0)
    src_k = _pad1(jnp.repeat(jnp.arange(N, dtype=jnp.int32), K), EPAD_K, 0)
    dst_k = _pad1(idx.reshape(-1), EPAD_K, N)
    w_k = _pad1(vals.reshape(-1), EPAD_K, 0.0)

    zeros_hid = jnp.zeros((NPAD, HID), jnp.float32)
    ones_hid = jnp.ones((N, HID), jnp.float32)
    degg = _agg_hid(ones_hid, dst_g, dst_g, w_g, zeros_hid)
    degk = _agg_hid_k(ones_hid, dst_k, dst_k, w_k, zeros_hid)
    dinvg, dinvk = _dinv_call(degg, degk)

    h = x
    ones = jnp.ones((N, 1), jnp.float32)
    for (W, b, sc, scb, dk, dkb, layer_a, agg_g, agg_k, combine, F) in (
        (W0, b0, s0, sb0, dk0, dkb0, _layer_a_1, _agg_hid, _agg_hid_k, _combine_1, HID),
        (W1, b1, s1, sb1, dk1, dkb1, _layer_a_2, _agg_out, _agg_out_k, _combine_2, OUT),
    ):
        h_aug = jnp.concatenate([h, ones], axis=1)
        top = jnp.concatenate([W, sc, dk], axis=1)
        bot = jnp.concatenate(
            [jnp.zeros((1, F), jnp.float32), scb.reshape(1, 1), dkb.reshape(1, 1)], axis=1)
        w_aug = jnp.concatenate([top, bot], axis=0)
        maug, mscg, msck = layer_a(h_aug, w_aug, dinvg, dinvk)
        zerosF = jnp.zeros((NPAD, F), jnp.float32)
        aggg = agg_g(mscg, src_g, dst_g, w_g, zerosF)
        aggk = agg_k(msck, src_k, dst_k, w_k, zerosF)
        h = combine(maug, aggg, aggk, dinvg, dinvk, b.reshape(1, F))
    return h


# speedup vs baseline: 5.2950x; 1.1324x over previous
import functools

import jax
import jax.numpy as jnp
from jax import lax
from jax.experimental import pallas as pl
from jax.experimental.pallas import tpu as pltpu
from jax.experimental.pallas import tpu_sc as plsc

N = 10000
NPAD = 10008
F_IN = 256
HID = 128
OUT = 64
K = 20
GAMMA = 0.01

ROW_BLK = 200
RB = 200
NC, NS = 2, 16
NW = NC * NS
CH = 128

E_G = 160000
E_K = N * K
EPAD_G = ((E_G + NW * CH - 1) // (NW * CH)) * NW * CH
EPAD_K = ((E_K + NW * CH - 1) // (NW * CH)) * NW * CH
PT_G = EPAD_G // NW
PT_K = EPAD_K // NW


def _knn_body(a_r_ref, a_all_ref, vals_ref, idx_ref):
    i = pl.program_id(0)
    sims = lax.dot_general(a_r_ref[...], a_all_ref[...], (((1,), (1,)), ((), ())),
                           preferred_element_type=jnp.float32)
    col = lax.broadcasted_iota(jnp.int32, sims.shape, 1)
    rowid = lax.broadcasted_iota(jnp.int32, sims.shape, 0) + i * ROW_BLK
    sims = jnp.where(col == rowid, 0.0, sims)
    vlist, ilist = [], []
    for _ in range(K):
        mx = jnp.max(sims, axis=1, keepdims=True)
        j = jnp.min(jnp.where(sims == mx, col, N), axis=1, keepdims=True)
        vlist.append(mx)
        ilist.append(j)
        sims = jnp.where(col == j, -1.0, sims)
    vals_ref[...] = jnp.concatenate(vlist, axis=1)
    idx_ref[...] = jnp.concatenate(ilist, axis=1)


def _knn_topk(a):
    return pl.pallas_call(
        _knn_body,
        grid=(N // ROW_BLK,),
        in_specs=[
            pl.BlockSpec((ROW_BLK, F_IN), lambda i: (i, 0)),
            pl.BlockSpec((N, F_IN), lambda i: (0, 0)),
        ],
        out_specs=[
            pl.BlockSpec((ROW_BLK, K), lambda i: (i, 0)),
            pl.BlockSpec((ROW_BLK, K), lambda i: (i, 0)),
        ],
        out_shape=[
            jax.ShapeDtypeStruct((N, K), jnp.float32),
            jax.ShapeDtypeStruct((N, K), jnp.int32),
        ],
    )(a, a)


PT16_G = EPAD_G // NS
PT16_K = EPAD_K // NS

_SC_PARAMS = pltpu.CompilerParams(needs_layout_passes=False,
                                  use_tc_tiling_on_sc=False)
_SC_MESH = plsc.VectorSubcoreMesh(core_axis_name="c", subcore_axis_name="s")


def _deg_dual_body(dst_g, w_g, dst_k, w_k, zeros16, out, dbuf, wbuf, rows16, acc):
    c = lax.axis_index("c")
    s = lax.axis_index("s")

    @pl.when(s == 0)
    def _():
        pltpu.sync_copy(zeros16, acc)
    plsc.subcore_barrier()

    def run(dst_ref, w_ref, per_tile):
        base = s * per_tile

        def step(i, carry):
            off = base + i * CH
            pltpu.sync_copy(dst_ref.at[pl.ds(off, CH)], dbuf)
            pltpu.sync_copy(w_ref.at[pl.ds(off, CH)], wbuf)

            def fill(e, cc):
                wv = plsc.load_gather(wbuf, [jnp.full((16,), e, jnp.int32)])
                rows16[e, :] = wv
                return cc
            lax.fori_loop(0, CH, fill, 0)
            pltpu.sync_copy(rows16, acc.at[dbuf], add=True)
            return carry
        lax.fori_loop(0, per_tile // CH, step, 0)

    @pl.when(c == 0)
    def _():
        run(dst_g, w_g, PT16_G)

    @pl.when(c == 1)
    def _():
        run(dst_k, w_k, PT16_K)
    plsc.subcore_barrier()

    @pl.when(s == 0)
    def _():
        pltpu.sync_copy(acc, out.at[c])


def _deg_dual(dst_g, w_g, dst_k, w_k, zeros16):
    f = pl.kernel(
        _deg_dual_body,
        out_type=jax.ShapeDtypeStruct((NC, NPAD, 16), jnp.float32),
        mesh=_SC_MESH,
        scratch_types=[
            pltpu.VMEM((CH,), jnp.int32),
            pltpu.VMEM((CH,), jnp.float32),
            pltpu.VMEM((CH, 16), jnp.float32),
            pltpu.VMEM_SHARED((NPAD, 16), jnp.float32),
        ],
        compiler_params=_SC_PARAMS,
    )
    return f(dst_g, w_g, dst_k, w_k, zeros16)


def _make_agg_dual(F):
    def body(mscg, msck, src_g, dst_g, w_g, src_k, dst_k, w_k, zerosF, out,
             sbuf, dbuf, wbuf, rows, acc, sem):
        c = lax.axis_index("c")
        s = lax.axis_index("s")

        @pl.when(s == 0)
        def _():
            pltpu.sync_copy(zerosF, acc)
        plsc.subcore_barrier()

        def run(msc, src, dst, w, per_tile):
            base = s * per_tile

            def step(i, carry):
                off = base + i * CH
                pltpu.sync_copy(src.at[pl.ds(off, CH)], sbuf)
                pltpu.sync_copy(dst.at[pl.ds(off, CH)], dbuf)
                pltpu.sync_copy(w.at[pl.ds(off, CH)], wbuf)
                pltpu.async_copy(msc.at[sbuf], rows, sem).wait()

                def scale(e, cc):
                    wv = plsc.load_gather(wbuf, [jnp.full((16,), e, jnp.int32)])
                    for f in range(F // 16):
                        sl = pl.ds(f * 16, 16)
                        rows[e, sl] = rows[e, sl] * wv
                    return cc
                lax.fori_loop(0, CH, scale, 0)
                pltpu.sync_copy(rows, acc.at[dbuf], add=True)
                return carry
            lax.fori_loop(0, per_tile // CH, step, 0)

        @pl.when(c == 0)
        def _():
            run(mscg, src_g, dst_g, w_g, PT16_G)

        @pl.when(c == 1)
        def _():
            run(msck, src_k, dst_k, w_k, PT16_K)
        plsc.subcore_barrier()

        @pl.when(s == 0)
        def _():
            pltpu.sync_copy(acc, out.at[c])

    def call(mscg, msck, src_g, dst_g, w_g, src_k, dst_k, w_k, zerosF):
        f = pl.kernel(
            body,
            out_type=jax.ShapeDtypeStruct((NC, NPAD, F), jnp.float32),
            mesh=_SC_MESH,
            scratch_types=[
                pltpu.VMEM((CH,), jnp.int32),
                pltpu.VMEM((CH,), jnp.int32),
                pltpu.VMEM((CH,), jnp.float32),
                pltpu.VMEM((CH, F), jnp.float32),
                pltpu.VMEM_SHARED((NPAD, F), jnp.float32),
                pltpu.SemaphoreType.DMA,
            ],
            compiler_params=_SC_PARAMS,
        )
        return f(mscg, msck, src_g, dst_g, w_g, src_k, dst_k, w_k, zerosF)

    return call


_agg_dual_hid = _make_agg_dual(HID)
_agg_dual_out = _make_agg_dual(OUT)


def _dinv_body(d_ref, og_ref, ok_ref):
    dg = d_ref[0, :, 0:1] + 1.0
    dk = d_ref[1, :, 0:1] + 1.0
    og_ref[...] = jnp.where(dg > 0, lax.rsqrt(dg), 0.0)
    ok_ref[...] = jnp.where(dk > 0, lax.rsqrt(dk), 0.0)


def _dinv_call(deg_both):
    return pl.pallas_call(
        _dinv_body,
        grid=(N // RB,),
        in_specs=[
            pl.BlockSpec((NC, RB, 16), lambda i: (0, i, 0)),
        ],
        out_specs=[
            pl.BlockSpec((RB, 1), lambda i: (i, 0)),
            pl.BlockSpec((RB, 1), lambda i: (i, 0)),
        ],
        out_shape=[
            jax.ShapeDtypeStruct((N, 1), jnp.float32),
            jax.ShapeDtypeStruct((N, 1), jnp.float32),
        ],
    )(deg_both)


def _make_layer_a(Fin1, F):
    def body(h_ref, w_ref, dg_ref, dk_ref, ma_ref, mg_ref, mk_ref):
        ma = jnp.dot(h_ref[...], w_ref[...], preferred_element_type=jnp.float32)
        ma_ref[...] = ma
        m = ma[:, :F]
        mg_ref[...] = m * dg_ref[...]
        mk_ref[...] = m * dk_ref[...]

    def call(h_aug, w_aug, dinvg, dinvk):
        return pl.pallas_call(
            body,
            grid=(N // RB,),
            in_specs=[
                pl.BlockSpec((RB, Fin1), lambda i: (i, 0)),
                pl.BlockSpec((Fin1, F + 2), lambda i: (0, 0)),
                pl.BlockSpec((RB, 1), lambda i: (i, 0)),
                pl.BlockSpec((RB, 1), lambda i: (i, 0)),
            ],
            out_specs=[
                pl.BlockSpec((RB, F + 2), lambda i: (i, 0)),
                pl.BlockSpec((RB, F), lambda i: (i, 0)),
                pl.BlockSpec((RB, F), lambda i: (i, 0)),
            ],
            out_shape=[
                jax.ShapeDtypeStruct((N, F + 2), jnp.float32),
                jax.ShapeDtypeStruct((N, F), jnp.float32),
                jax.ShapeDtypeStruct((N, F), jnp.float32),
            ],
        )(h_aug, w_aug, dinvg, dinvk)

    return call


_layer_a_1 = _make_layer_a(F_IN + 1, HID)
_layer_a_2 = _make_layer_a(HID + 1, OUT)


def _make_combine(F):
    def body(ma_ref, agg_ref, dg_ref, dk_ref, b_ref, out_ref):
        ma = ma_ref[...]
        m = ma[:, :F]
        s = jax.nn.sigmoid(ma[:, F:F + 1])
        hdk = ma[:, F + 1:F + 2]
        dg = dg_ref[...]
        dk = dk_ref[...]
        b = b_ref[...]
        g = dg * agg_ref[0] + dg * dg * m + b
        kn = dk * agg_ref[1] + dk * dk * m + b
        out_ref[...] = s * g + (1.0 - s) * kn + GAMMA * hdk * (m + b)

    def call(maug, agg_both, dinvg, dinvk, b):
        return pl.pallas_call(
            body,
            grid=(N // RB,),
            in_specs=[
                pl.BlockSpec((RB, F + 2), lambda i: (i, 0)),
                pl.BlockSpec((NC, RB, F), lambda i: (0, i, 0)),
                pl.BlockSpec((RB, 1), lambda i: (i, 0)),
                pl.BlockSpec((RB, 1), lambda i: (i, 0)),
                pl.BlockSpec((1, F), lambda i: (0, 0)),
            ],
            out_specs=pl.BlockSpec((RB, F), lambda i: (i, 0)),
            out_shape=jax.ShapeDtypeStruct((N, F), jnp.float32),
        )(maug, agg_both, dinvg, dinvk, b)

    return call


_combine_1 = _make_combine(HID)
_combine_2 = _make_combine(OUT)


def _pad1(a, n, val):
    return jnp.concatenate([a, jnp.full((n - a.shape[0],), val, a.dtype)])


def kernel(x, edge_index, edge_weight, W0, b0, W1, b1, s0, sb0, dk0, dkb0, s1, sb1, dk1, dkb1):
    xb = (x != 0).astype(jnp.float32)
    a = xb / jnp.linalg.norm(xb, axis=1, keepdims=True)
    vals, idx = _knn_topk(a)

    ei = edge_index.astype(jnp.int32)
    src_g = _pad1(ei[0], EPAD_G, 0)
    dst_g = _pad1(ei[1], EPAD_G, N)
    w_g = _pad1(edge_weight.astype(jnp.float32), EPAD_G, 0.0)
    src_k = _pad1(jnp.repeat(jnp.arange(N, dtype=jnp.int32), K), EPAD_K, 0)
    dst_k = _pad1(idx.reshape(-1), EPAD_K, N)
    w_k = _pad1(vals.reshape(-1), EPAD_K, 0.0)

    zeros16 = jnp.zeros((NPAD, 16), jnp.float32)
    deg_both = _deg_dual(dst_g, w_g, dst_k, w_k, zeros16)
    dinvg, dinvk = _dinv_call(deg_both)

    h = x
    ones = jnp.ones((N, 1), jnp.float32)
    for (W, b, sc, scb, dk, dkb, layer_a, agg_dual, combine, F) in (
        (W0, b0, s0, sb0, dk0, dkb0, _layer_a_1, _agg_dual_hid, _combine_1, HID),
        (W1, b1, s1, sb1, dk1, dkb1, _layer_a_2, _agg_dual_out, _combine_2, OUT),
    ):
        h_aug = jnp.concatenate([h, ones], axis=1)
        top = jnp.concatenate([W, sc, dk], axis=1)
        bot = jnp.concatenate(
            [jnp.zeros((1, F), jnp.float32), scb.reshape(1, 1), dkb.reshape(1, 1)], axis=1)
        w_aug = jnp.concatenate([top, bot], axis=0)
        maug, mscg, msck = layer_a(h_aug, w_aug, dinvg, dinvk)
        zerosF = jnp.zeros((NPAD, F), jnp.float32)
        agg_both = agg_dual(mscg, msck, src_g, dst_g, w_g, src_k, dst_k, w_k, zerosF)
        h = combine(maug, agg_both, dinvg, dinvk, b.reshape(1, F))
    return h
